# baseline (speedup 1.0000x reference)
import functools

import jax
import jax.numpy as jnp
from jax import lax
from jax.experimental import pallas as pl
from jax.experimental.pallas import tpu as pltpu
from jax.experimental.pallas import tpu_sc as plsc

_B = 16384
_F = 26
_D = 32
_E = 64
_N = _B * _F
_V = 1000000
_NC = 2
_NS = 16
_NW = _NC * _NS
_ROWS_PER_W = _N // _NW
_SPW = 128
_STREAMS_PER_W = _ROWS_PER_W // _SPW
_G = 4
_GROWS = _G * _SPW
_NG = _STREAMS_PER_W // _G

_VB = 8192
_VH = _V // 2


def _proj_body(t_ref, w_ref, b_ref, o_ref):
    r_lo = lax.dot_general(
        t_ref[:, 0, :], w_ref[...],
        dimension_numbers=(((0,), (0,)), ((), ())),
        preferred_element_type=jnp.float32,
    ) + b_ref[...]
    r_hi = lax.dot_general(
        t_ref[:, 1, :], w_ref[...],
        dimension_numbers=(((0,), (0,)), ((), ())),
        preferred_element_type=jnp.float32,
    ) + b_ref[...]
    o_ref[:, :_E] = r_lo
    o_ref[:, _E:] = r_hi


@jax.jit
def _project_table(t3, W, b2):
    grid = (_VH + _VB - 1) // _VB
    return pl.pallas_call(
        _proj_body,
        grid=(grid,),
        in_specs=[
            pl.BlockSpec((_D, 2, _VB), lambda i: (0, 0, i)),
            pl.BlockSpec((_D, _E), lambda i: (0, 0)),
            pl.BlockSpec((1, _E), lambda i: (0, 0)),
        ],
        out_specs=pl.BlockSpec((_VB, 2 * _E), lambda i: (i, 0)),
        out_shape=jax.ShapeDtypeStruct((_VH, 2 * _E), jnp.float32),
    )(t3, W, b2)


def _gather_body(p_hbm, idx_hbm, out_hbm, idx_v, buf0, buf1,
                 gsem0, gsem1, wsem0, wsem1):
    wid = lax.axis_index("s") * _NC + lax.axis_index("c")
    pltpu.sync_copy(idx_hbm.at[pl.ds(wid * _STREAMS_PER_W, _STREAMS_PER_W)],
                    idx_v)
    base = wid * _ROWS_PER_W

    def fire(g, buf, gsem):
        for j in range(_G):
            pltpu.async_copy(
                p_hbm.at[idx_v.at[g * _G + j]],
                buf.at[pl.ds(j * _SPW, _SPW)],
                gsem,
            )

    def drain(buf, gsem):
        for j in range(_G):
            pltpu.make_async_copy(
                p_hbm.at[idx_v.at[j]],
                buf.at[pl.ds(j * _SPW, _SPW)],
                gsem,
            ).wait()

    fire(0, buf0, gsem0)

    def step(g2, carry):
        g = 2 * g2
        drain(buf0, gsem0)
        @pl.when(g2 > 0)
        def _():
            pltpu.make_async_copy(
                buf1, out_hbm.at[pl.ds(base, _GROWS)], wsem1
            ).wait()
        fire(g + 1, buf1, gsem1)
        pltpu.async_copy(
            buf0, out_hbm.at[pl.ds(base + g * _GROWS, _GROWS)], wsem0
        )
        drain(buf1, gsem1)
        pltpu.make_async_copy(
            buf0, out_hbm.at[pl.ds(base, _GROWS)], wsem0
        ).wait()
        @pl.when(g2 < _NG // 2 - 1)
        def _():
            fire(g + 2, buf0, gsem0)
            pltpu.async_copy(
                buf1, out_hbm.at[pl.ds(base + (g + 1) * _GROWS, _GROWS)],
                wsem1,
            )

        @pl.when(g2 == _NG // 2 - 1)
        def _():
            pltpu.sync_copy(
                buf1, out_hbm.at[pl.ds(base + (g + 1) * _GROWS, _GROWS)]
            )
        return carry

    lax.fori_loop(0, _NG // 2, step, 0)


@jax.jit
def _gather(p64, idx2d):
    mesh = plsc.VectorSubcoreMesh(core_axis_name="c", subcore_axis_name="s")
    return pl.kernel(
        _gather_body,
        mesh=mesh,
        out_type=jax.ShapeDtypeStruct((_N, _E), jnp.float32),
        scratch_types=[
            pltpu.VMEM((_STREAMS_PER_W, _SPW), jnp.int32),
            pltpu.VMEM((_GROWS, _E), jnp.float32),
            pltpu.VMEM((_GROWS, _E), jnp.float32),
            pltpu.SemaphoreType.DMA,
            pltpu.SemaphoreType.DMA,
            pltpu.SemaphoreType.DMA,
            pltpu.SemaphoreType.DMA,
        ],
        compiler_params=pltpu.CompilerParams(use_tc_tiling_on_sc=False),
    )(p64, idx2d)


def kernel(indexes, table, W, b):
    v = indexes.astype(jnp.int32)
    idx = (2 * lax.rem(v, _VH) + v // _VH).reshape(_NW * _STREAMS_PER_W, _SPW)
    t3 = table.T.reshape(_D, 2, _VH)
    p = _project_table(t3, W, b.reshape(1, _E))
    p64 = p.reshape(_V, _E)
    out = _gather(p64, idx)
    return out.reshape(_B, _F, _E)


# speedup vs baseline: 1.8374x; 1.8374x over previous
import functools

import jax
import jax.numpy as jnp
from jax import lax
from jax.experimental import pallas as pl
from jax.experimental.pallas import tpu as pltpu
from jax.experimental.pallas import tpu_sc as plsc

_B = 16384
_F = 26
_D = 32
_E = 64
_N = _B * _F
_V = 1000000
_NC = 2
_NS = 16
_NW = _NC * _NS
_ROWS_PER_W = _N // _NW
_SPW = 128
_STREAMS_PER_W = _ROWS_PER_W // _SPW
_G = 4
_GROWS = _G * _SPW
_NG = _STREAMS_PER_W // _G

_VB = 8192
_KB = 62
_K = _KB * _VB
_TLB = (_V + _VB - 1) // _VB - 1


def _proj_body(tlo_ref, thi_ref, w_ref, b_ref, o_ref):
    r_lo = lax.dot_general(
        tlo_ref[...], w_ref[...],
        dimension_numbers=(((0,), (0,)), ((), ())),
        preferred_element_type=jnp.float32,
    ) + b_ref[...]
    r_hi = lax.dot_general(
        thi_ref[...], w_ref[...],
        dimension_numbers=(((0,), (0,)), ((), ())),
        preferred_element_type=jnp.float32,
    ) + b_ref[...]
    o_ref[:, :_E] = r_lo
    o_ref[:, _E:] = r_hi


@jax.jit
def _project_table(tableT, W, b2):
    return pl.pallas_call(
        _proj_body,
        grid=(_KB,),
        in_specs=[
            pl.BlockSpec((_D, _VB), lambda i: (0, i)),
            pl.BlockSpec((_D, _VB),
                         lambda i: (0, jnp.minimum(i + _KB, _TLB))),
            pl.BlockSpec((_D, _E), lambda i: (0, 0)),
            pl.BlockSpec((1, _E), lambda i: (0, 0)),
        ],
        out_specs=pl.BlockSpec((_VB, 2 * _E), lambda i: (i, 0)),
        out_shape=jax.ShapeDtypeStruct((_K, 2 * _E), jnp.float32),
    )(tableT, tableT, W, b2)


def _gather_body(p_hbm, idx_hbm, out_hbm, idx_v, buf0, buf1,
                 gsem0, gsem1, wsem0, wsem1):
    wid = lax.axis_index("s") * _NC + lax.axis_index("c")
    pltpu.sync_copy(idx_hbm.at[pl.ds(wid * _STREAMS_PER_W, _STREAMS_PER_W)],
                    idx_v)
    base = wid * _ROWS_PER_W

    def fire(g, buf, gsem):
        for j in range(_G):
            pltpu.async_copy(
                p_hbm.at[idx_v.at[g * _G + j]],
                buf.at[pl.ds(j * _SPW, _SPW)],
                gsem,
            )

    def drain(buf, gsem):
        for j in range(_G):
            pltpu.make_async_copy(
                p_hbm.at[idx_v.at[j]],
                buf.at[pl.ds(j * _SPW, _SPW)],
                gsem,
            ).wait()

    fire(0, buf0, gsem0)

    def step(g2, carry):
        g = 2 * g2
        drain(buf0, gsem0)
        @pl.when(g2 > 0)
        def _():
            pltpu.make_async_copy(
                buf1, out_hbm.at[pl.ds(base, _GROWS)], wsem1
            ).wait()
        fire(g + 1, buf1, gsem1)
        pltpu.async_copy(
            buf0, out_hbm.at[pl.ds(base + g * _GROWS, _GROWS)], wsem0
        )
        drain(buf1, gsem1)
        pltpu.make_async_copy(
            buf0, out_hbm.at[pl.ds(base, _GROWS)], wsem0
        ).wait()
        @pl.when(g2 < _NG // 2 - 1)
        def _():
            fire(g + 2, buf0, gsem0)
            pltpu.async_copy(
                buf1, out_hbm.at[pl.ds(base + (g + 1) * _GROWS, _GROWS)],
                wsem1,
            )

        @pl.when(g2 == _NG // 2 - 1)
        def _():
            pltpu.sync_copy(
                buf1, out_hbm.at[pl.ds(base + (g + 1) * _GROWS, _GROWS)]
            )
        return carry

    lax.fori_loop(0, _NG // 2, step, 0)


@jax.jit
def _gather(p64, idx2d):
    mesh = plsc.VectorSubcoreMesh(core_axis_name="c", subcore_axis_name="s")
    return pl.kernel(
        _gather_body,
        mesh=mesh,
        out_type=jax.ShapeDtypeStruct((_N, _E), jnp.float32),
        scratch_types=[
            pltpu.VMEM((_STREAMS_PER_W, _SPW), jnp.int32),
            pltpu.VMEM((_GROWS, _E), jnp.float32),
            pltpu.VMEM((_GROWS, _E), jnp.float32),
            pltpu.SemaphoreType.DMA,
            pltpu.SemaphoreType.DMA,
            pltpu.SemaphoreType.DMA,
            pltpu.SemaphoreType.DMA,
        ],
        compiler_params=pltpu.CompilerParams(use_tc_tiling_on_sc=False),
    )(p64, idx2d)


def kernel(indexes, table, W, b):
    v = indexes.astype(jnp.int32)
    idx = jnp.where(v < _K, 2 * v, 2 * (v - _K) + 1)
    idx = idx.reshape(_NW * _STREAMS_PER_W, _SPW)
    p = _project_table(table.T, W, b.reshape(1, _E))
    p64 = p.reshape(2 * _K, _E)
    out = _gather(p64, idx)
    return out.reshape(_B, _F, _E)


# speedup vs baseline: 1.8445x; 1.0038x over previous
import functools

import jax
import jax.numpy as jnp
from jax import lax
from jax.experimental import pallas as pl
from jax.experimental.pallas import tpu as pltpu
from jax.experimental.pallas import tpu_sc as plsc

_B = 16384
_F = 26
_D = 32
_E = 64
_N = _B * _F
_V = 1000000
_NC = 2
_NS = 16
_NW = _NC * _NS
_ROWS_PER_W = _N // _NW
_SPW = 128
_STREAMS_PER_W = _ROWS_PER_W // _SPW
_G = 4
_GROWS = _G * _SPW
_NG = _STREAMS_PER_W // _G

_VB = 16384
_KB = 31
_K = _KB * _VB
_TLB = (_V + _VB - 1) // _VB - 1


def _proj_body(tlo_ref, thi_ref, w_ref, b_ref, o_ref):
    r_lo = lax.dot_general(
        tlo_ref[...], w_ref[...],
        dimension_numbers=(((0,), (0,)), ((), ())),
        preferred_element_type=jnp.float32,
    ) + b_ref[...]
    r_hi = lax.dot_general(
        thi_ref[...], w_ref[...],
        dimension_numbers=(((0,), (0,)), ((), ())),
        preferred_element_type=jnp.float32,
    ) + b_ref[...]
    o_ref[:, :_E] = r_lo
    o_ref[:, _E:] = r_hi


@jax.jit
def _project_table(tableT, W, b2):
    return pl.pallas_call(
        _proj_body,
        grid=(_KB,),
        in_specs=[
            pl.BlockSpec((_D, _VB), lambda i: (0, i)),
            pl.BlockSpec((_D, _VB),
                         lambda i: (0, jnp.minimum(i + _KB, _TLB))),
            pl.BlockSpec((_D, _E), lambda i: (0, 0)),
            pl.BlockSpec((1, _E), lambda i: (0, 0)),
        ],
        out_specs=pl.BlockSpec((_VB, 2 * _E), lambda i: (i, 0)),
        out_shape=jax.ShapeDtypeStruct((_K, 2 * _E), jnp.float32),
    )(tableT, tableT, W, b2)


def _gather_body(p_hbm, idx_hbm, out_hbm, idx_v, buf0, buf1,
                 gsem0, gsem1, wsem0, wsem1):
    wid = lax.axis_index("s") * _NC + lax.axis_index("c")
    pltpu.sync_copy(idx_hbm.at[pl.ds(wid * _STREAMS_PER_W, _STREAMS_PER_W)],
                    idx_v)
    base = wid * _ROWS_PER_W

    def fire(g, buf, gsem):
        for j in range(_G):
            pltpu.async_copy(
                p_hbm.at[idx_v.at[g * _G + j]],
                buf.at[pl.ds(j * _SPW, _SPW)],
                gsem,
            )

    def drain(buf, gsem):
        for j in range(_G):
            pltpu.make_async_copy(
                p_hbm.at[idx_v.at[j]],
                buf.at[pl.ds(j * _SPW, _SPW)],
                gsem,
            ).wait()

    fire(0, buf0, gsem0)

    def step(g2, carry):
        g = 2 * g2
        drain(buf0, gsem0)
        @pl.when(g2 > 0)
        def _():
            pltpu.make_async_copy(
                buf1, out_hbm.at[pl.ds(base, _GROWS)], wsem1
            ).wait()
        fire(g + 1, buf1, gsem1)
        pltpu.async_copy(
            buf0, out_hbm.at[pl.ds(base + g * _GROWS, _GROWS)], wsem0
        )
        drain(buf1, gsem1)
        pltpu.make_async_copy(
            buf0, out_hbm.at[pl.ds(base, _GROWS)], wsem0
        ).wait()
        @pl.when(g2 < _NG // 2 - 1)
        def _():
            fire(g + 2, buf0, gsem0)
            pltpu.async_copy(
                buf1, out_hbm.at[pl.ds(base + (g + 1) * _GROWS, _GROWS)],
                wsem1,
            )

        @pl.when(g2 == _NG // 2 - 1)
        def _():
            pltpu.sync_copy(
                buf1, out_hbm.at[pl.ds(base + (g + 1) * _GROWS, _GROWS)]
            )
        return carry

    lax.fori_loop(0, _NG // 2, step, 0)


@jax.jit
def _gather(p64, idx2d):
    mesh = plsc.VectorSubcoreMesh(core_axis_name="c", subcore_axis_name="s")
    return pl.kernel(
        _gather_body,
        mesh=mesh,
        out_type=jax.ShapeDtypeStruct((_N, _E), jnp.float32),
        scratch_types=[
            pltpu.VMEM((_STREAMS_PER_W, _SPW), jnp.int32),
            pltpu.VMEM((_GROWS, _E), jnp.float32),
            pltpu.VMEM((_GROWS, _E), jnp.float32),
            pltpu.SemaphoreType.DMA,
            pltpu.SemaphoreType.DMA,
            pltpu.SemaphoreType.DMA,
            pltpu.SemaphoreType.DMA,
        ],
        compiler_params=pltpu.CompilerParams(use_tc_tiling_on_sc=False),
    )(p64, idx2d)


def kernel(indexes, table, W, b):
    v = indexes.astype(jnp.int32)
    idx = jnp.where(v < _K, 2 * v, 2 * (v - _K) + 1)
    idx = idx.reshape(_NW * _STREAMS_PER_W, _SPW)
    p = _project_table(table.T, W, b.reshape(1, _E))
    p64 = p.reshape(2 * _K, _E)
    out = _gather(p64, idx)
    return out.reshape(_B, _F, _E)
